# baseline (speedup 1.0000x reference)
import functools

import jax
import jax.numpy as jnp
from jax import lax
from jax.experimental import pallas as pl
from jax.experimental.pallas import tpu as pltpu
from jax.experimental.pallas import tpu_sc as plsc

B = 16384
D = 32
NC = 2
NS = 16
L = 16
NW = NC * NS
BPW = B // NW


def _sc_scores(user, pos, neg, ui_t, up_t, ii_t, ip_t):
    mesh = plsc.VectorSubcoreMesh(core_axis_name="c", subcore_axis_name="s")

    @functools.partial(
        pl.kernel,
        out_type=[jax.ShapeDtypeStruct((B,), jnp.float32)] * 4,
        mesh=mesh,
        scratch_types=[
            pltpu.VMEM((BPW,), jnp.int32),
            pltpu.VMEM((BPW,), jnp.int32),
            pltpu.VMEM((BPW,), jnp.int32),
            pltpu.VMEM((D, BPW), jnp.float32),
            pltpu.VMEM((D, BPW), jnp.float32),
            pltpu.VMEM((D, BPW), jnp.float32),
            pltpu.VMEM((D, BPW), jnp.float32),
            pltpu.VMEM((D, BPW), jnp.float32),
            pltpu.VMEM((D, BPW), jnp.float32),
            pltpu.VMEM((BPW,), jnp.float32),
            pltpu.VMEM((BPW,), jnp.float32),
            pltpu.VMEM((BPW,), jnp.float32),
            pltpu.VMEM((BPW,), jnp.float32),
            pltpu.SemaphoreType.DMA,
        ],
        compiler_params=pltpu.CompilerParams(
            needs_layout_passes=False, use_tc_tiling_on_sc=False),
    )
    def body(user_h, pos_h, neg_h, uit_h, upt_h, iit_h, ipt_h,
             o_pint, o_nint, o_ppop, o_npop,
             uidx, pidx, nidx, cui, cup, cpi, cpp, cni, cnp,
             s_pint, s_nint, s_ppop, s_npop, sem):
        wid = lax.axis_index("s") * NC + lax.axis_index("c")
        base = wid * BPW

        pltpu.sync_copy(user_h.at[pl.ds(base, BPW)], uidx)
        pltpu.sync_copy(pos_h.at[pl.ds(base, BPW)], pidx)
        pltpu.sync_copy(neg_h.at[pl.ds(base, BPW)], nidx)

        handles = []
        for c in range(D):
            handles.append(pltpu.async_copy(uit_h.at[c].at[uidx], cui.at[c], sem))
            handles.append(pltpu.async_copy(upt_h.at[c].at[uidx], cup.at[c], sem))
            handles.append(pltpu.async_copy(iit_h.at[c].at[pidx], cpi.at[c], sem))
            handles.append(pltpu.async_copy(ipt_h.at[c].at[pidx], cpp.at[c], sem))
            handles.append(pltpu.async_copy(iit_h.at[c].at[nidx], cni.at[c], sem))
            handles.append(pltpu.async_copy(ipt_h.at[c].at[nidx], cnp.at[c], sem))
        for h in handles:
            h.wait()

        def blk_body(blk, _):
            off = blk * L
            zero = jnp.zeros((L,), jnp.float32)
            a_pint, a_nint, a_ppop, a_npop = zero, zero, zero, zero
            for c in range(D):
                ui = cui[c, pl.ds(off, L)]
                up = cup[c, pl.ds(off, L)]
                pi = cpi[c, pl.ds(off, L)]
                pp = cpp[c, pl.ds(off, L)]
                ni = cni[c, pl.ds(off, L)]
                np_ = cnp[c, pl.ds(off, L)]
                a_pint = a_pint + ui * pi
                a_nint = a_nint + ui * ni
                a_ppop = a_ppop + up * pp
                a_npop = a_npop + up * np_
            s_pint[pl.ds(off, L)] = a_pint
            s_nint[pl.ds(off, L)] = a_nint
            s_ppop[pl.ds(off, L)] = a_ppop
            s_npop[pl.ds(off, L)] = a_npop
            return _

        lax.fori_loop(0, BPW // L, blk_body, None)

        pltpu.sync_copy(s_pint, o_pint.at[pl.ds(base, BPW)])
        pltpu.sync_copy(s_nint, o_nint.at[pl.ds(base, BPW)])
        pltpu.sync_copy(s_ppop, o_ppop.at[pl.ds(base, BPW)])
        pltpu.sync_copy(s_npop, o_npop.at[pl.ds(base, BPW)])

    return body(user, pos, neg, ui_t, up_t, ii_t, ip_t)


def _tc_loss_body(pint_ref, nint_ref, ppop_ref, npop_ref, mask_ref, out_ref):
    m = jnp.clip(mask_ref[...], 0.0, 1.0)

    def bpr(x):
        sig = 1.0 / (1.0 + jnp.exp(-x))
        return -jnp.log(sig + 1e-08)

    pint = pint_ref[...]
    nint = nint_ref[...]
    ppop = ppop_ref[...]
    npop = npop_ref[...]
    total = (
        jnp.sum(bpr(pint - nint) * m)
        + jnp.sum(bpr(npop - ppop) * (1.0 - m))
        + jnp.sum(bpr(ppop - npop) * m)
    )
    out_ref[0, 0] = total / B


def kernel(user, pos, neg, mask, pos_period, neg_period,
           users_int, users_pop, items_int, items_pop):
    del pos_period, neg_period
    pint, nint, ppop, npop = _sc_scores(
        user.astype(jnp.int32), pos.astype(jnp.int32), neg.astype(jnp.int32),
        users_int.T, users_pop.T, items_int.T, items_pop.T)

    shape2d = (B // 128, 128)
    loss = pl.pallas_call(
        _tc_loss_body,
        out_shape=jax.ShapeDtypeStruct((1, 1), jnp.float32),
        out_specs=pl.BlockSpec(memory_space=pltpu.SMEM),
    )(pint.reshape(shape2d), nint.reshape(shape2d),
      ppop.reshape(shape2d), npop.reshape(shape2d),
      mask.astype(jnp.float32).reshape(shape2d))
    return loss[0, 0]
